# baseline (speedup 1.0000x reference)
import jax
import jax.numpy as jnp
import numpy as np
from jax import lax
from jax.experimental import pallas as pl
from jax.experimental.pallas import tpu as pltpu
from jax.experimental.pallas import tpu_sc as plsc

_NC = 2
_NS = 16
_NW = _NC * _NS
_L = 16

_N = 10000
_E = 160000
_EP = 160256
_EW = _EP // _NW
_NP = 10240
_RW = _NP // _NW
_D = 512
_DH = _D // 2
_CAP = 5888

_SQRT2 = np.sqrt(2.0).astype(np.float32)
_INV_SQRT_C = np.float32(1.0 / np.sqrt(512.0))


def _gelu(t):
    return t * 0.5 * (1.0 + lax.erf(t / _SQRT2))


_RB = 1000


def _tc1_body(x_ref, wq, bq, wk, bk, wv, bv, ws, bs, q_o, k_o, vl_o, vh_o, s_o):
    xb = x_ref[...]
    q_o[...] = (jnp.dot(xb, wq[...], preferred_element_type=jnp.float32)
                + bq[...]).astype(jnp.bfloat16)
    k_o[...] = (jnp.dot(xb, wk[...], preferred_element_type=jnp.float32)
                + bk[...]).astype(jnp.bfloat16)
    v = jnp.dot(xb, wv[...], preferred_element_type=jnp.float32) + bv[...]
    vl_o[...] = v[:, :_DH]
    vh_o[...] = v[:, _DH:]
    s_o[...] = jnp.dot(xb, ws[...], preferred_element_type=jnp.float32) + bs[...]


def _tc_proj1(x, Wq, bq, Wk, bk, Wv, bv, Ws, bs):
    din = x.shape[1]
    grid = _N // _RB
    w_spec = pl.BlockSpec((din, _D), lambda i: (0, 0))
    b_spec = pl.BlockSpec((_D,), lambda i: (0,))
    o_spec = pl.BlockSpec((_RB, _D), lambda i: (i, 0))
    oh_spec = pl.BlockSpec((_RB, _DH), lambda i: (i, 0))
    return pl.pallas_call(
        _tc1_body,
        grid=(grid,),
        in_specs=[pl.BlockSpec((_RB, din), lambda i: (i, 0)),
                  w_spec, b_spec, w_spec, b_spec, w_spec, b_spec, w_spec, b_spec],
        out_specs=[o_spec, o_spec, oh_spec, oh_spec, o_spec],
        out_shape=[jax.ShapeDtypeStruct((_N, _D), jnp.bfloat16),
                   jax.ShapeDtypeStruct((_N, _D), jnp.bfloat16),
                   jax.ShapeDtypeStruct((_N, _DH), jnp.float32),
                   jax.ShapeDtypeStruct((_N, _DH), jnp.float32),
                   jax.ShapeDtypeStruct((_N, _D), jnp.float32)],
    )(x, Wq, bq, Wk, bk, Wv, bv, Ws, bs)


def _tc2_body(al_ref, ah_ref, s1_ref, wq, bq, wk, bk, wv, bv, ws, bs,
              q_o, k_o, vl_o, vh_o, s_o):
    attn = jnp.concatenate([al_ref[...], ah_ref[...]], axis=-1)
    h = _gelu(attn + s1_ref[...])
    q_o[...] = (jnp.dot(h, wq[...], preferred_element_type=jnp.float32)
                + bq[...]).astype(jnp.bfloat16)
    k_o[...] = (jnp.dot(h, wk[...], preferred_element_type=jnp.float32)
                + bk[...]).astype(jnp.bfloat16)
    v = jnp.dot(h, wv[...], preferred_element_type=jnp.float32) + bv[...]
    vl_o[...] = v[:, :_DH]
    vh_o[...] = v[:, _DH:]
    s_o[...] = jnp.dot(h, ws[...], preferred_element_type=jnp.float32) + bs[...]


def _tc_proj2(alo, ahi, s1, Wq, bq, Wk, bk, Wv, bv, Ws, bs):
    grid = _N // _RB
    w_spec = pl.BlockSpec((_D, _D), lambda i: (0, 0))
    b_spec = pl.BlockSpec((_D,), lambda i: (0,))
    io_spec = pl.BlockSpec((_RB, _D), lambda i: (i, 0))
    ih_spec = pl.BlockSpec((_RB, _DH), lambda i: (i, 0))
    return pl.pallas_call(
        _tc2_body,
        grid=(grid,),
        in_specs=[ih_spec, ih_spec, io_spec,
                  w_spec, b_spec, w_spec, b_spec, w_spec, b_spec, w_spec, b_spec],
        out_specs=[io_spec, io_spec, ih_spec, ih_spec, io_spec],
        out_shape=[jax.ShapeDtypeStruct((_N, _D), jnp.bfloat16),
                   jax.ShapeDtypeStruct((_N, _D), jnp.bfloat16),
                   jax.ShapeDtypeStruct((_N, _DH), jnp.float32),
                   jax.ShapeDtypeStruct((_N, _DH), jnp.float32),
                   jax.ShapeDtypeStruct((_N, _D), jnp.float32)],
    )(alo, ahi, s1, Wq, bq, Wk, bk, Wv, bv, Ws, bs)


def _tc3_body(al_ref, ah_ref, s2_ref, x_ref, wh, bh, wx, bx, o_ref):
    attn = jnp.concatenate([al_ref[...], ah_ref[...]], axis=-1)
    h2 = _gelu(_gelu(attn + s2_ref[...]))
    hs = jnp.dot(h2, wh[...], preferred_element_type=jnp.float32) + bh[...]
    xs = jnp.dot(x_ref[...], wx[...], preferred_element_type=jnp.float32) + bx[...]
    o_ref[...] = _gelu(hs + xs)


def _tc_final(alo, ahi, s2, x, Wh, bh, Wx, bx):
    dout = Wx.shape[1]
    din = x.shape[1]
    grid = _N // _RB
    return pl.pallas_call(
        _tc3_body,
        grid=(grid,),
        in_specs=[pl.BlockSpec((_RB, _DH), lambda i: (i, 0)),
                  pl.BlockSpec((_RB, _DH), lambda i: (i, 0)),
                  pl.BlockSpec((_RB, _D), lambda i: (i, 0)),
                  pl.BlockSpec((_RB, din), lambda i: (i, 0)),
                  pl.BlockSpec((_D, dout), lambda i: (0, 0)),
                  pl.BlockSpec((dout,), lambda i: (0,)),
                  pl.BlockSpec((din, dout), lambda i: (0, 0)),
                  pl.BlockSpec((dout,), lambda i: (0,))],
        out_specs=pl.BlockSpec((_RB, dout), lambda i: (i, 0)),
        out_shape=jax.ShapeDtypeStruct((_N, dout), jnp.float32),
    )(alo, ahi, s2, x, Wh, bh, Wx, bx)


_GA = _EW // _L


_AB = 4


def _sc_alpha_body(q_h, k_h, srcp_h, dstp_h, ea_h, dpart_h,
                   srcb, dstb, eab, qb0, kb0, qb1, kb1, qb2, kb2, qb3, kb3,
                   accb, zb, denom_sp, sq0, sk0, sq1, sk1, sq2, sk2, sq3, sk3):
    c = lax.axis_index("c")
    s = lax.axis_index("s")
    wid = c * _NS + s
    ebase = wid * _EW
    pltpu.sync_copy(srcp_h.at[pl.ds(ebase, _EW)], srcb)
    pltpu.sync_copy(dstp_h.at[pl.ds(ebase, _EW)], dstb)

    def _zzb(i, carry):
        zb[pl.ds(i * _L, _L)] = jnp.zeros((_L,), jnp.float32)
        return carry

    lax.fori_loop(0, 640 // _L, _zzb, 0)
    pltpu.sync_copy(zb, denom_sp.at[pl.ds(s * 640, 640)])

    zeros16 = jnp.zeros((_L,), jnp.float32)
    rowi = lax.iota(jnp.int32, _L)

    qbufs = (qb0, qb1, qb2, qb3)
    kbufs = (kb0, kb1, kb2, kb3)
    sqs = (sq0, sq1, sq2, sq3)
    sks = (sk0, sk1, sk2, sk3)

    def _issue(g, b):
        dvec = dstb[pl.ds(g * _L, _L)]
        svec = srcb[pl.ds(g * _L, _L)]
        pltpu.async_copy(q_h.at[dvec], qbufs[b], sqs[b])
        pltpu.async_copy(k_h.at[svec], kbufs[b], sks[b])

    for b in range(_AB):
        _issue(b, b)

    def _make_group(b):
        qb, kb, sq, sk = qbufs[b], kbufs[b], sqs[b], sks[b]

        def _group(g, carry):
            pltpu.make_async_copy(q_h.at[pl.ds(0, _L)], qb, sq).wait()
            pltpu.make_async_copy(k_h.at[pl.ds(0, _L)], kb, sk).wait()
            for e in range(_L):
                acc = zeros16
                for j in range(_D // (2 * _L)):
                    qa, qc = plsc.unpack(
                        plsc.bitcast(qb[e, pl.ds(j * _L, _L)], jnp.bfloat16),
                        format=plsc.PackFormat.INTERLEAVED)
                    ka, kc = plsc.unpack(
                        plsc.bitcast(kb[e, pl.ds(j * _L, _L)], jnp.bfloat16),
                        format=plsc.PackFormat.INTERLEAVED)
                    acc = acc + qa * ka + qc * kc
                accb[pl.ds(e * _L, _L)] = acc
            alpha = zeros16
            for j in range(_L):
                alpha = alpha + plsc.load_gather(
                    accb, [rowi * _L + jnp.full((_L,), j, jnp.int32)])
            ea = jnp.exp(alpha * _INV_SQRT_C)
            valid = (ebase + g * _L + rowi) < _E
            ea = jnp.where(valid, ea, 0.0)
            eab[pl.ds(g * _L, _L)] = ea
            @pl.when(g + _AB < _GA)
            def _():
                _issue(g + _AB, b)
            return carry

        return _group

    gfns = [_make_group(b) for b in range(_AB)]

    def _quad(p, carry):
        for b in range(_AB):
            carry = gfns[b](p * _AB + b, carry)
        return carry

    lax.fori_loop(0, _GA // _AB, _quad, 0)
    for g in range(_GA - _GA % _AB, _GA):
        gfns[g % _AB](g, 0)

    pltpu.sync_copy(eab, ea_h.at[pl.ds(ebase, _EW)])

    plsc.subcore_barrier()
    pltpu.sync_copy(eab, denom_sp.at[dstb], add=True)
    plsc.subcore_barrier()
    pltpu.sync_copy(denom_sp.at[pl.ds(s * 640, 640)],
                    dpart_h.at[pl.ds(c * _NP + s * 640, 640)])


def _sc_alpha(q, k, srcp, dstp):
    mesh = plsc.VectorSubcoreMesh(core_axis_name="c", subcore_axis_name="s",
                                  num_cores=_NC, num_subcores=_NS)
    f = pl.kernel(
        _sc_alpha_body,
        out_type=(jax.ShapeDtypeStruct((_EP,), jnp.float32),
                  jax.ShapeDtypeStruct((_NC * _NP,), jnp.float32)),
        mesh=mesh,
        compiler_params=pltpu.CompilerParams(needs_layout_passes=False),
        scratch_types=[
            pltpu.VMEM((_EW,), jnp.int32),
            pltpu.VMEM((_EW,), jnp.int32),
            pltpu.VMEM((_EW,), jnp.float32),
            pltpu.VMEM((_L, _D // 2), jnp.float32),
            pltpu.VMEM((_L, _D // 2), jnp.float32),
            pltpu.VMEM((_L, _D // 2), jnp.float32),
            pltpu.VMEM((_L, _D // 2), jnp.float32),
            pltpu.VMEM((_L, _D // 2), jnp.float32),
            pltpu.VMEM((_L, _D // 2), jnp.float32),
            pltpu.VMEM((_L, _D // 2), jnp.float32),
            pltpu.VMEM((_L, _D // 2), jnp.float32),
            pltpu.VMEM((_L * _L,), jnp.float32),
            pltpu.VMEM((640,), jnp.float32),
            pltpu.VMEM_SHARED((16 * 640,), jnp.float32),
            pltpu.SemaphoreType.DMA,
            pltpu.SemaphoreType.DMA,
            pltpu.SemaphoreType.DMA,
            pltpu.SemaphoreType.DMA,
            pltpu.SemaphoreType.DMA,
            pltpu.SemaphoreType.DMA,
            pltpu.SemaphoreType.DMA,
            pltpu.SemaphoreType.DMA,
        ],
    )
    return f(q, k, srcp, dstp)


_SCN = _EP // _EW


def _sc_agg_body(vl_h, vh_h, ea_h, dpart_h, srcp_h, dstp_h, alo_h, ahi_h,
                 srcb, dstb, eab, csrc, cdl, cea, dnb, vb0, vb1, vb2, block,
                 sv0, sv1, sv2):
    c = lax.axis_index("c")
    s = lax.axis_index("s")
    wid = c * _NS + s
    lo = wid * _RW

    zeros16 = jnp.zeros((_L,), jnp.float32)
    izeros16 = jnp.zeros((_L,), jnp.int32)
    rowi = lax.iota(jnp.int32, _L)

    pltpu.sync_copy(dpart_h.at[pl.ds(lo, _RW)], dnb.at[pl.ds(0, _RW)])
    pltpu.sync_copy(dpart_h.at[pl.ds(_NP + lo, _RW)], dnb.at[pl.ds(_RW, _RW)])

    def _inv(i, carry):
        dnb[pl.ds(i * _L, _L)] = 1.0 / (dnb[pl.ds(i * _L, _L)]
                                        + dnb[pl.ds(_RW + i * _L, _L)]
                                        + 1e-16)
        return carry

    lax.fori_loop(0, _RW // _L, _inv, 0)

    def _scan_chunk(ch, cur):
        base = ch * _EW
        pltpu.sync_copy(srcp_h.at[pl.ds(base, _EW)], srcb)
        pltpu.sync_copy(dstp_h.at[pl.ds(base, _EW)], dstb)
        pltpu.sync_copy(ea_h.at[pl.ds(base, _EW)], eab)

        def _cgroup(g, cur):
            dvec = dstb[pl.ds(g * _L, _L)]
            svec = srcb[pl.ds(g * _L, _L)]
            ea16 = eab[pl.ds(g * _L, _L)]
            m = (dvec >= lo) & (dvec < lo + _RW)
            cur_c = jnp.minimum(cur, _CAP)
            plsc.store_compressed(csrc.at[pl.ds(cur_c, _L)], svec, mask=m)
            plsc.store_compressed(cdl.at[pl.ds(cur_c, _L)], dvec - lo, mask=m)
            plsc.store_compressed(cea.at[pl.ds(cur_c, _L)], ea16, mask=m)
            cnt = plsc.all_reduce_population_count(m)
            if cnt.ndim:
                cnt = jnp.max(cnt)
            return cur + cnt

        return lax.fori_loop(0, _EW // _L, _cgroup, cur)

    ctot = lax.fori_loop(0, _SCN, _scan_chunk, 0)
    ctot = jnp.minimum(ctot, _CAP)
    csrc[pl.ds(ctot, _L)] = izeros16
    cdl[pl.ds(ctot, _L)] = izeros16
    cea[pl.ds(ctot, _L)] = zeros16
    nch = (ctot + _L - 1) // _L

    def _wg(g, carry):
        cd16 = cdl[pl.ds(g * _L, _L)]
        cea[pl.ds(g * _L, _L)] = (cea[pl.ds(g * _L, _L)]
                                  * plsc.load_gather(dnb, [cd16]))
        return carry

    lax.fori_loop(0, nch, _wg, 0)

    vbufs = (vb0, vb1, vb2)
    svs = (sv0, sv1, sv2)

    for half in range(2):
        v_h = (vl_h, vh_h)[half]
        a_h = (alo_h, ahi_h)[half]

        def _zb(i, carry):
            block[pl.ds(i * _L, _L)] = zeros16
            return carry

        lax.fori_loop(0, _RW * _DH // _L, _zb, 0)

        def _vissue(ci, b):
            idx = csrc[pl.ds(ci * _L, _L)]
            pltpu.async_copy(v_h.at[idx], vbufs[b], svs[b])

        @pl.when(nch > 0)
        def _():
            _vissue(0, 0)

        @pl.when(nch > 1)
        def _():
            _vissue(1, 1)

        @pl.when(nch > 2)
        def _():
            _vissue(2, 2)

        def _make_chunk(b):
            vb, sv = vbufs[b], svs[b]

            def _chunk(ci, carry):
                pltpu.make_async_copy(v_h.at[pl.ds(0, _L)], vb, sv).wait()
                cd16 = cdl[pl.ds(ci * _L, _L)]
                rb16 = cd16 * _DH
                for e in range(_L):
                    wv = plsc.load_gather(
                        cea, [jnp.full((_L,), ci * _L + e, jnp.int32)])
                    rb = jnp.max(jnp.where(rowi == e, rb16, 0))
                    for j in range(_DH // _L):
                        plsc.addupdate(block.at[pl.ds(rb + j * _L, _L)],
                                       vb[e, pl.ds(j * _L, _L)] * wv)
                @pl.when(ci + 3 < nch)
                def _():
                    _vissue(ci + 3, b)
                return carry

            return _chunk

        chs = [_make_chunk(b) for b in range(3)]

        def _chtriple(p, carry):
            carry = chs[0](p * 3, carry)

            @pl.when(p * 3 + 1 < nch)
            def _():
                chs[1](p * 3 + 1, 0)

            @pl.when(p * 3 + 2 < nch)
            def _():
                chs[2](p * 3 + 2, 0)
            return carry

        lax.fori_loop(0, (nch + 2) // 3, _chtriple, 0)
        pltpu.sync_copy(block, a_h.at[pl.ds(lo * _DH, _RW * _DH)])


def _sc_agg(vl, vh, ea, dpart, srcp, dstp):
    mesh = plsc.VectorSubcoreMesh(core_axis_name="c", subcore_axis_name="s",
                                  num_cores=_NC, num_subcores=_NS)
    f = pl.kernel(
        _sc_agg_body,
        out_type=(jax.ShapeDtypeStruct((_NP * _DH,), jnp.float32),
                  jax.ShapeDtypeStruct((_NP * _DH,), jnp.float32)),
        mesh=mesh,
        compiler_params=pltpu.CompilerParams(needs_layout_passes=False),
        scratch_types=[
            pltpu.VMEM((_EW,), jnp.int32),
            pltpu.VMEM((_EW,), jnp.int32),
            pltpu.VMEM((_EW,), jnp.float32),
            pltpu.VMEM((_CAP + _L,), jnp.int32),
            pltpu.VMEM((_CAP + _L,), jnp.int32),
            pltpu.VMEM((_CAP + _L,), jnp.float32),
            pltpu.VMEM((2 * _RW,), jnp.float32),
            pltpu.VMEM((_L, _DH), jnp.float32),
            pltpu.VMEM((_L, _DH), jnp.float32),
            pltpu.VMEM((_L, _DH), jnp.float32),
            pltpu.VMEM((_RW * _DH,), jnp.float32),
            pltpu.SemaphoreType.DMA,
            pltpu.SemaphoreType.DMA,
            pltpu.SemaphoreType.DMA,
        ],
    )
    return f(vl, vh, ea, dpart, srcp, dstp)


def _pack32(a16):
    n, d = a16.shape
    return lax.bitcast_convert_type(a16.reshape(n, d // 2, 2), jnp.float32)


def kernel(x, edge_index, edge_attr, Wq1, bq1, Wk1, bk1, Wv1, bv1, Ws1, bs1,
           Wq2, bq2, Wk2, bk2, Wv2, bv2, Ws2, bs2, Wx, bx, Wh, bh):
    del edge_attr
    src = edge_index[0]
    dst = edge_index[1]
    pad = _EP - _E
    srcp = jnp.concatenate([src, jnp.zeros((pad,), jnp.int32)])
    dstp = jnp.concatenate([dst, jnp.full((pad,), _N - 1, jnp.int32)])

    q1, k1, vl1, vh1, s1 = _tc_proj1(x, Wq1, bq1, Wk1, bk1, Wv1, bv1, Ws1, bs1)
    ea1, dp1 = _sc_alpha(_pack32(q1), _pack32(k1), srcp, dstp)
    al1, ah1 = _sc_agg(vl1, vh1, ea1, dp1, srcp, dstp)
    al1 = al1.reshape(_NP, _DH)[:_N]
    ah1 = ah1.reshape(_NP, _DH)[:_N]
    q2, k2, vl2, vh2, s2 = _tc_proj2(al1, ah1, s1, Wq2, bq2, Wk2, bk2,
                                     Wv2, bv2, Ws2, bs2)
    ea2, dp2 = _sc_alpha(_pack32(q2), _pack32(k2), srcp, dstp)
    al2, ah2 = _sc_agg(vl2, vh2, ea2, dp2, srcp, dstp)
    al2 = al2.reshape(_NP, _DH)[:_N]
    ah2 = ah2.reshape(_NP, _DH)[:_N]
    return _tc_final(al2, ah2, s2, x, Wh, bh, Wx, bx)


# speedup vs baseline: 1.0266x; 1.0266x over previous
import jax
import jax.numpy as jnp
import numpy as np
from jax import lax
from jax.experimental import pallas as pl
from jax.experimental.pallas import tpu as pltpu
from jax.experimental.pallas import tpu_sc as plsc

_NC = 2
_NS = 16
_NW = _NC * _NS
_L = 16

_N = 10000
_E = 160000
_EP = 160256
_EW = _EP // _NW
_NP = 10240
_RW = _NP // _NW
_D = 512
_DH = _D // 2
_CAP = 7680

_SQRT2 = np.sqrt(2.0).astype(np.float32)
_INV_SQRT_C = np.float32(1.0 / np.sqrt(512.0))


def _gelu(t):
    return t * 0.5 * (1.0 + lax.erf(t / _SQRT2))


_RB = 1000


def _tc1_body(x_ref, wq, bq, wk, bk, wv, bv, ws, bs, q_o, k_o, vl_o, vh_o, s_o):
    xb = x_ref[...]
    q_o[...] = (jnp.dot(xb, wq[...], preferred_element_type=jnp.float32)
                + bq[...]).astype(jnp.bfloat16)
    k_o[...] = (jnp.dot(xb, wk[...], preferred_element_type=jnp.float32)
                + bk[...]).astype(jnp.bfloat16)
    v = jnp.dot(xb, wv[...], preferred_element_type=jnp.float32) + bv[...]
    vl_o[...] = v[:, :_DH]
    vh_o[...] = v[:, _DH:]
    s_o[...] = jnp.dot(xb, ws[...], preferred_element_type=jnp.float32) + bs[...]


def _tc_proj1(x, Wq, bq, Wk, bk, Wv, bv, Ws, bs):
    din = x.shape[1]
    grid = _N // _RB
    w_spec = pl.BlockSpec((din, _D), lambda i: (0, 0))
    b_spec = pl.BlockSpec((_D,), lambda i: (0,))
    o_spec = pl.BlockSpec((_RB, _D), lambda i: (i, 0))
    oh_spec = pl.BlockSpec((_RB, _DH), lambda i: (i, 0))
    return pl.pallas_call(
        _tc1_body,
        grid=(grid,),
        in_specs=[pl.BlockSpec((_RB, din), lambda i: (i, 0)),
                  w_spec, b_spec, w_spec, b_spec, w_spec, b_spec, w_spec, b_spec],
        out_specs=[o_spec, o_spec, oh_spec, oh_spec, o_spec],
        out_shape=[jax.ShapeDtypeStruct((_N, _D), jnp.bfloat16),
                   jax.ShapeDtypeStruct((_N, _D), jnp.bfloat16),
                   jax.ShapeDtypeStruct((_N, _DH), jnp.float32),
                   jax.ShapeDtypeStruct((_N, _DH), jnp.float32),
                   jax.ShapeDtypeStruct((_N, _D), jnp.float32)],
    )(x, Wq, bq, Wk, bk, Wv, bv, Ws, bs)


def _tc2_body(al_ref, ah_ref, s1_ref, wq, bq, wk, bk, wv, bv, ws, bs,
              q_o, k_o, vl_o, vh_o, s_o):
    attn = jnp.concatenate([al_ref[...], ah_ref[...]], axis=-1)
    h = _gelu(attn + s1_ref[...])
    q_o[...] = (jnp.dot(h, wq[...], preferred_element_type=jnp.float32)
                + bq[...]).astype(jnp.bfloat16)
    k_o[...] = (jnp.dot(h, wk[...], preferred_element_type=jnp.float32)
                + bk[...]).astype(jnp.bfloat16)
    v = jnp.dot(h, wv[...], preferred_element_type=jnp.float32) + bv[...]
    vl_o[...] = v[:, :_DH]
    vh_o[...] = v[:, _DH:]
    s_o[...] = jnp.dot(h, ws[...], preferred_element_type=jnp.float32) + bs[...]


def _tc_proj2(alo, ahi, s1, Wq, bq, Wk, bk, Wv, bv, Ws, bs):
    grid = _N // _RB
    w_spec = pl.BlockSpec((_D, _D), lambda i: (0, 0))
    b_spec = pl.BlockSpec((_D,), lambda i: (0,))
    io_spec = pl.BlockSpec((_RB, _D), lambda i: (i, 0))
    ih_spec = pl.BlockSpec((_RB, _DH), lambda i: (i, 0))
    return pl.pallas_call(
        _tc2_body,
        grid=(grid,),
        in_specs=[ih_spec, ih_spec, io_spec,
                  w_spec, b_spec, w_spec, b_spec, w_spec, b_spec, w_spec, b_spec],
        out_specs=[io_spec, io_spec, ih_spec, ih_spec, io_spec],
        out_shape=[jax.ShapeDtypeStruct((_N, _D), jnp.bfloat16),
                   jax.ShapeDtypeStruct((_N, _D), jnp.bfloat16),
                   jax.ShapeDtypeStruct((_N, _DH), jnp.float32),
                   jax.ShapeDtypeStruct((_N, _DH), jnp.float32),
                   jax.ShapeDtypeStruct((_N, _D), jnp.float32)],
    )(alo, ahi, s1, Wq, bq, Wk, bk, Wv, bv, Ws, bs)


def _tc3_body(al_ref, ah_ref, s2_ref, x_ref, wh, bh, wx, bx, o_ref):
    attn = jnp.concatenate([al_ref[...], ah_ref[...]], axis=-1)
    h2 = _gelu(_gelu(attn + s2_ref[...]))
    hs = jnp.dot(h2, wh[...], preferred_element_type=jnp.float32) + bh[...]
    xs = jnp.dot(x_ref[...], wx[...], preferred_element_type=jnp.float32) + bx[...]
    o_ref[...] = _gelu(hs + xs)


def _tc_final(alo, ahi, s2, x, Wh, bh, Wx, bx):
    dout = Wx.shape[1]
    din = x.shape[1]
    grid = _N // _RB
    return pl.pallas_call(
        _tc3_body,
        grid=(grid,),
        in_specs=[pl.BlockSpec((_RB, _DH), lambda i: (i, 0)),
                  pl.BlockSpec((_RB, _DH), lambda i: (i, 0)),
                  pl.BlockSpec((_RB, _D), lambda i: (i, 0)),
                  pl.BlockSpec((_RB, din), lambda i: (i, 0)),
                  pl.BlockSpec((_D, dout), lambda i: (0, 0)),
                  pl.BlockSpec((dout,), lambda i: (0,)),
                  pl.BlockSpec((din, dout), lambda i: (0, 0)),
                  pl.BlockSpec((dout,), lambda i: (0,))],
        out_specs=pl.BlockSpec((_RB, dout), lambda i: (i, 0)),
        out_shape=jax.ShapeDtypeStruct((_N, dout), jnp.float32),
    )(alo, ahi, s2, x, Wh, bh, Wx, bx)


_GA = _EW // _L


_AB = 4


def _sc_alpha_body(q_h, k_h, srcp_h, dstp_h, ea_h, dpart_h,
                   srcb, dstb, eab, qb0, kb0, qb1, kb1, qb2, kb2, qb3, kb3,
                   accb, zb, denom_sp, sq0, sk0, sq1, sk1, sq2, sk2, sq3, sk3):
    c = lax.axis_index("c")
    s = lax.axis_index("s")
    wid = c * _NS + s
    ebase = wid * _EW
    pltpu.sync_copy(srcp_h.at[pl.ds(ebase, _EW)], srcb)
    pltpu.sync_copy(dstp_h.at[pl.ds(ebase, _EW)], dstb)

    def _zzb(i, carry):
        zb[pl.ds(i * _L, _L)] = jnp.zeros((_L,), jnp.float32)
        return carry

    lax.fori_loop(0, 640 // _L, _zzb, 0)
    pltpu.sync_copy(zb, denom_sp.at[pl.ds(s * 640, 640)])

    zeros16 = jnp.zeros((_L,), jnp.float32)
    rowi = lax.iota(jnp.int32, _L)

    qbufs = (qb0, qb1, qb2, qb3)
    kbufs = (kb0, kb1, kb2, kb3)
    sqs = (sq0, sq1, sq2, sq3)
    sks = (sk0, sk1, sk2, sk3)

    def _issue(g, b):
        dvec = dstb[pl.ds(g * _L, _L)]
        svec = srcb[pl.ds(g * _L, _L)]
        pltpu.async_copy(q_h.at[dvec], qbufs[b], sqs[b])
        pltpu.async_copy(k_h.at[svec], kbufs[b], sks[b])

    for b in range(_AB):
        _issue(b, b)

    def _make_group(b):
        qb, kb, sq, sk = qbufs[b], kbufs[b], sqs[b], sks[b]

        def _group(g, carry):
            pltpu.make_async_copy(q_h.at[pl.ds(0, _L)], qb, sq).wait()
            pltpu.make_async_copy(k_h.at[pl.ds(0, _L)], kb, sk).wait()
            for e in range(_L):
                acc = zeros16
                for j in range(_D // (2 * _L)):
                    qa, qc = plsc.unpack(
                        plsc.bitcast(qb[e, pl.ds(j * _L, _L)], jnp.bfloat16),
                        format=plsc.PackFormat.INTERLEAVED)
                    ka, kc = plsc.unpack(
                        plsc.bitcast(kb[e, pl.ds(j * _L, _L)], jnp.bfloat16),
                        format=plsc.PackFormat.INTERLEAVED)
                    acc = acc + qa * ka + qc * kc
                accb[pl.ds(e * _L, _L)] = acc
            alpha = zeros16
            for j in range(_L):
                alpha = alpha + plsc.load_gather(
                    accb, [rowi * _L + jnp.full((_L,), j, jnp.int32)])
            ea = jnp.exp(alpha * _INV_SQRT_C)
            valid = (ebase + g * _L + rowi) < _E
            ea = jnp.where(valid, ea, 0.0)
            eab[pl.ds(g * _L, _L)] = ea
            @pl.when(g + _AB < _GA)
            def _():
                _issue(g + _AB, b)
            return carry

        return _group

    gfns = [_make_group(b) for b in range(_AB)]

    def _quad(p, carry):
        for b in range(_AB):
            carry = gfns[b](p * _AB + b, carry)
        return carry

    lax.fori_loop(0, _GA // _AB, _quad, 0)
    for g in range(_GA - _GA % _AB, _GA):
        gfns[g % _AB](g, 0)

    pltpu.sync_copy(eab, ea_h.at[pl.ds(ebase, _EW)])

    plsc.subcore_barrier()
    pltpu.sync_copy(eab, denom_sp.at[dstb], add=True)
    plsc.subcore_barrier()
    pltpu.sync_copy(denom_sp.at[pl.ds(s * 640, 640)],
                    dpart_h.at[pl.ds(c * _NP + s * 640, 640)])


def _sc_alpha(q, k, srcp, dstp):
    mesh = plsc.VectorSubcoreMesh(core_axis_name="c", subcore_axis_name="s",
                                  num_cores=_NC, num_subcores=_NS)
    f = pl.kernel(
        _sc_alpha_body,
        out_type=(jax.ShapeDtypeStruct((_EP,), jnp.float32),
                  jax.ShapeDtypeStruct((_NC * _NP,), jnp.float32)),
        mesh=mesh,
        compiler_params=pltpu.CompilerParams(needs_layout_passes=False),
        scratch_types=[
            pltpu.VMEM((_EW,), jnp.int32),
            pltpu.VMEM((_EW,), jnp.int32),
            pltpu.VMEM((_EW,), jnp.float32),
            pltpu.VMEM((_L, _D // 2), jnp.float32),
            pltpu.VMEM((_L, _D // 2), jnp.float32),
            pltpu.VMEM((_L, _D // 2), jnp.float32),
            pltpu.VMEM((_L, _D // 2), jnp.float32),
            pltpu.VMEM((_L, _D // 2), jnp.float32),
            pltpu.VMEM((_L, _D // 2), jnp.float32),
            pltpu.VMEM((_L, _D // 2), jnp.float32),
            pltpu.VMEM((_L, _D // 2), jnp.float32),
            pltpu.VMEM((_L * _L,), jnp.float32),
            pltpu.VMEM((640,), jnp.float32),
            pltpu.VMEM_SHARED((16 * 640,), jnp.float32),
            pltpu.SemaphoreType.DMA,
            pltpu.SemaphoreType.DMA,
            pltpu.SemaphoreType.DMA,
            pltpu.SemaphoreType.DMA,
            pltpu.SemaphoreType.DMA,
            pltpu.SemaphoreType.DMA,
            pltpu.SemaphoreType.DMA,
            pltpu.SemaphoreType.DMA,
        ],
    )
    return f(q, k, srcp, dstp)


_SCN = _EP // _EW


def _sc_agg_body(vl_h, vh_h, ea_h, dpart_h, srcp_h, dstp_h, alo_h, ahi_h,
                 srcb, dstb, eab, csrc, cdl, cea, dnb, vb0, vb1, block,
                 sv0, sv1):
    c = lax.axis_index("c")
    s = lax.axis_index("s")
    wid = c * _NS + s
    lo = wid * _RW

    zeros16 = jnp.zeros((_L,), jnp.float32)
    izeros16 = jnp.zeros((_L,), jnp.int32)
    rowi = lax.iota(jnp.int32, _L)

    pltpu.sync_copy(dpart_h.at[pl.ds(lo, _RW)], dnb.at[pl.ds(0, _RW)])
    pltpu.sync_copy(dpart_h.at[pl.ds(_NP + lo, _RW)], dnb.at[pl.ds(_RW, _RW)])

    def _inv(i, carry):
        dnb[pl.ds(i * _L, _L)] = 1.0 / (dnb[pl.ds(i * _L, _L)]
                                        + dnb[pl.ds(_RW + i * _L, _L)]
                                        + 1e-16)
        return carry

    lax.fori_loop(0, _RW // _L, _inv, 0)

    def _scan_chunk(ch, cur):
        base = ch * _EW
        pltpu.sync_copy(srcp_h.at[pl.ds(base, _EW)], srcb)
        pltpu.sync_copy(dstp_h.at[pl.ds(base, _EW)], dstb)
        pltpu.sync_copy(ea_h.at[pl.ds(base, _EW)], eab)

        def _cgroup(g, cur):
            dvec = dstb[pl.ds(g * _L, _L)]
            svec = srcb[pl.ds(g * _L, _L)]
            ea16 = eab[pl.ds(g * _L, _L)]
            m = (dvec >= lo) & (dvec < lo + _RW)
            cur_c = jnp.minimum(cur, _CAP)
            plsc.store_compressed(csrc.at[pl.ds(cur_c, _L)], svec, mask=m)
            plsc.store_compressed(cdl.at[pl.ds(cur_c, _L)], dvec - lo, mask=m)
            plsc.store_compressed(cea.at[pl.ds(cur_c, _L)], ea16, mask=m)
            cnt = plsc.all_reduce_population_count(m)
            if cnt.ndim:
                cnt = jnp.max(cnt)
            return cur + cnt

        return lax.fori_loop(0, _EW // _L, _cgroup, cur)

    ctot = lax.fori_loop(0, _SCN, _scan_chunk, 0)
    ctot = jnp.minimum(ctot, _CAP)
    csrc[pl.ds(ctot, _L)] = izeros16
    cdl[pl.ds(ctot, _L)] = izeros16
    cea[pl.ds(ctot, _L)] = zeros16
    nch = (ctot + _L - 1) // _L

    def _wg(g, carry):
        cd16 = cdl[pl.ds(g * _L, _L)]
        cea[pl.ds(g * _L, _L)] = (cea[pl.ds(g * _L, _L)]
                                  * plsc.load_gather(dnb, [cd16]))
        return carry

    lax.fori_loop(0, nch, _wg, 0)

    vbufs = (vb0, vb1)
    svs = (sv0, sv1)

    for half in range(2):
        v_h = (vl_h, vh_h)[half]
        a_h = (alo_h, ahi_h)[half]

        def _zb(i, carry):
            block[pl.ds(i * _L, _L)] = zeros16
            return carry

        lax.fori_loop(0, _RW * _DH // _L, _zb, 0)

        def _vissue(ci, b):
            idx = csrc[pl.ds(ci * _L, _L)]
            pltpu.async_copy(v_h.at[idx], vbufs[b], svs[b])

        @pl.when(nch > 0)
        def _():
            _vissue(0, 0)

        @pl.when(nch > 1)
        def _():
            _vissue(1, 1)

        def _make_chunk(b):
            vb, sv = vbufs[b], svs[b]

            def _chunk(ci, carry):
                pltpu.make_async_copy(v_h.at[pl.ds(0, _L)], vb, sv).wait()
                cd16 = cdl[pl.ds(ci * _L, _L)]
                rb16 = cd16 * _DH
                for e in range(_L):
                    wv = plsc.load_gather(
                        cea, [jnp.full((_L,), ci * _L + e, jnp.int32)])
                    rb = jnp.max(jnp.where(rowi == e, rb16, 0))
                    for j in range(_DH // _L):
                        plsc.addupdate(block.at[pl.ds(rb + j * _L, _L)],
                                       vb[e, pl.ds(j * _L, _L)] * wv)
                @pl.when(ci + 2 < nch)
                def _():
                    _vissue(ci + 2, b)
                return carry

            return _chunk

        ch0 = _make_chunk(0)
        ch1 = _make_chunk(1)

        def _chpair(p, carry):
            carry = ch0(p * 2, carry)

            @pl.when(p * 2 + 1 < nch)
            def _():
                ch1(p * 2 + 1, 0)
            return carry

        lax.fori_loop(0, (nch + 1) // 2, _chpair, 0)
        pltpu.sync_copy(block, a_h.at[pl.ds(lo * _DH, _RW * _DH)])


def _sc_agg(vl, vh, ea, dpart, srcp, dstp):
    mesh = plsc.VectorSubcoreMesh(core_axis_name="c", subcore_axis_name="s",
                                  num_cores=_NC, num_subcores=_NS)
    f = pl.kernel(
        _sc_agg_body,
        out_type=(jax.ShapeDtypeStruct((_NP * _DH,), jnp.float32),
                  jax.ShapeDtypeStruct((_NP * _DH,), jnp.float32)),
        mesh=mesh,
        compiler_params=pltpu.CompilerParams(needs_layout_passes=False),
        scratch_types=[
            pltpu.VMEM((_EW,), jnp.int32),
            pltpu.VMEM((_EW,), jnp.int32),
            pltpu.VMEM((_EW,), jnp.float32),
            pltpu.VMEM((_CAP + _L,), jnp.int32),
            pltpu.VMEM((_CAP + _L,), jnp.int32),
            pltpu.VMEM((_CAP + _L,), jnp.float32),
            pltpu.VMEM((2 * _RW,), jnp.float32),
            pltpu.VMEM((_L, _DH), jnp.float32),
            pltpu.VMEM((_L, _DH), jnp.float32),
            pltpu.VMEM((_RW * _DH,), jnp.float32),
            pltpu.SemaphoreType.DMA,
            pltpu.SemaphoreType.DMA,
        ],
    )
    return f(vl, vh, ea, dpart, srcp, dstp)


def _pack32(a16):
    n, d = a16.shape
    return lax.bitcast_convert_type(a16.reshape(n, d // 2, 2), jnp.float32)


def kernel(x, edge_index, edge_attr, Wq1, bq1, Wk1, bk1, Wv1, bv1, Ws1, bs1,
           Wq2, bq2, Wk2, bk2, Wv2, bv2, Ws2, bs2, Wx, bx, Wh, bh):
    del edge_attr
    src = edge_index[0]
    dst = edge_index[1]
    pad = _EP - _E
    srcp = jnp.concatenate([src, jnp.zeros((pad,), jnp.int32)])
    dstp = jnp.concatenate([dst, jnp.full((pad,), _N - 1, jnp.int32)])

    q1, k1, vl1, vh1, s1 = _tc_proj1(x, Wq1, bq1, Wk1, bk1, Wv1, bv1, Ws1, bs1)
    ea1, dp1 = _sc_alpha(_pack32(q1), _pack32(k1), srcp, dstp)
    al1, ah1 = _sc_agg(vl1, vh1, ea1, dp1, srcp, dstp)
    al1 = al1.reshape(_NP, _DH)[:_N]
    ah1 = ah1.reshape(_NP, _DH)[:_N]
    q2, k2, vl2, vh2, s2 = _tc_proj2(al1, ah1, s1, Wq2, bq2, Wk2, bk2,
                                     Wv2, bv2, Ws2, bs2)
    ea2, dp2 = _sc_alpha(_pack32(q2), _pack32(k2), srcp, dstp)
    al2, ah2 = _sc_agg(vl2, vh2, ea2, dp2, srcp, dstp)
    al2 = al2.reshape(_NP, _DH)[:_N]
    ah2 = ah2.reshape(_NP, _DH)[:_N]
    return _tc_final(al2, ah2, s2, x, Wh, bh, Wx, bx)
